# Initial kernel scaffold; baseline (speedup 1.0000x reference)
import jax
import jax.numpy as jnp
from jax import lax
from jax.experimental import pallas as pl
from jax.experimental.pallas import tpu as pltpu
from jax.experimental.pallas import tpu_sc as plsc

N_DOCS = 1000000
B = 16384
L = 50
NC = 2
NS = 16
NW = NC * NS
CHUNK = (B // NW) * L


def _body(x_hbm, table_hbm, y_hbm, exam_hbm, rel_hbm, idx_v, rel_v, exam_v, y_v, sem):
    wid = lax.axis_index("s") * NC + lax.axis_index("c")
    base = wid * CHUNK

    pltpu.sync_copy(x_hbm.at[pl.ds(base, CHUNK)], idx_v)
    pltpu.async_copy(table_hbm.at[idx_v], rel_v, sem).wait()

    iota = lax.iota(jnp.int32, 16)
    rows_per_grp = 16
    n_grp = (B // NW) // rows_per_grp

    def grp(bb, carry):
        p0 = bb * (rows_per_grp * L) + iota * L
        ex = jnp.full((16,), 1.0, dtype=jnp.float32)
        for l in range(L):
            pv = p0 + l
            e = plsc.load_gather(rel_v, [pv])
            r = 1.0 / (1.0 + jnp.exp(-e))
            plsc.store_scatter(rel_v, [pv], r)
            plsc.store_scatter(exam_v, [pv], ex)
            plsc.store_scatter(y_v, [pv], ex * r)
            ex = ex * (1.0 - r)
        return carry

    lax.fori_loop(0, n_grp, grp, 0)

    pltpu.sync_copy(y_v, y_hbm.at[pl.ds(base, CHUNK)])
    pltpu.sync_copy(exam_v, exam_hbm.at[pl.ds(base, CHUNK)])
    pltpu.sync_copy(rel_v, rel_hbm.at[pl.ds(base, CHUNK)])


def kernel(x, table):
    x_flat = jnp.reshape(x.astype(jnp.int32), (B * L,))
    t_flat = jnp.reshape(table, (N_DOCS,))

    f32 = jnp.float32
    call = pl.kernel(
        _body,
        out_type=(
            jax.ShapeDtypeStruct((B * L,), f32),
            jax.ShapeDtypeStruct((B * L,), f32),
            jax.ShapeDtypeStruct((B * L,), f32),
        ),
        mesh=plsc.VectorSubcoreMesh(core_axis_name="c", subcore_axis_name="s"),
        scratch_types=[
            pltpu.VMEM((CHUNK,), jnp.int32),
            pltpu.VMEM((CHUNK,), f32),
            pltpu.VMEM((CHUNK,), f32),
            pltpu.VMEM((CHUNK,), f32),
            pltpu.SemaphoreType.DMA,
        ],
    )
    y, exam, rel = call(x_flat, t_flat)
    return (
        jnp.reshape(y, (B, L)),
        jnp.reshape(exam, (B, L)),
        jnp.reshape(rel, (B, L)),
    )


# speedup vs baseline: 1.0778x; 1.0778x over previous
import jax
import jax.numpy as jnp
from jax import lax
from jax.experimental import pallas as pl
from jax.experimental.pallas import tpu as pltpu
from jax.experimental.pallas import tpu_sc as plsc

N_DOCS = 1000000
B = 16384
L = 50
NC = 2
NS = 16
NW = NC * NS
CHUNK = (B // NW) * L


def _body(x_hbm, table_hbm, y_hbm, exam_hbm, rel_hbm, idx_v, rel_v, exam_v, y_v, sem):
    wid = lax.axis_index("s") * NC + lax.axis_index("c")
    base = wid * CHUNK

    pltpu.sync_copy(x_hbm.at[pl.ds(base, CHUNK)], idx_v)
    pltpu.async_copy(table_hbm.at[idx_v], rel_v, sem).wait()

    iota = lax.iota(jnp.int32, 16)
    rows_per_grp = 16
    n_grp = (B // NW) // rows_per_grp

    def grp(bb, carry):
        p0 = bb * (rows_per_grp * L) + iota * L
        ex = jnp.full((16,), 1.0, dtype=jnp.float32)
        for l in range(L):
            pv = p0 + l
            e = plsc.load_gather(rel_v, [pv])
            r = 1.0 / (1.0 + jnp.exp(-e))
            plsc.store_scatter(rel_v, [pv], r)
            plsc.store_scatter(exam_v, [pv], ex)
            plsc.store_scatter(y_v, [pv], ex * r)
            ex = ex * (1.0 - r)
        return carry

    lax.fori_loop(0, n_grp, grp, 0)

    pltpu.sync_copy(y_v, y_hbm.at[pl.ds(base, CHUNK)])
    pltpu.sync_copy(exam_v, exam_hbm.at[pl.ds(base, CHUNK)])
    pltpu.sync_copy(rel_v, rel_hbm.at[pl.ds(base, CHUNK)])


def kernel(x, table):
    x_flat = jnp.reshape(x.astype(jnp.int32), (B * L,))
    t_flat = jnp.reshape(table, (N_DOCS,))

    f32 = jnp.float32
    call = pl.kernel(
        _body,
        out_type=(
            jax.ShapeDtypeStruct((B * L,), f32),
            jax.ShapeDtypeStruct((B * L,), f32),
            jax.ShapeDtypeStruct((B * L,), f32),
        ),
        mesh=plsc.VectorSubcoreMesh(core_axis_name="c", subcore_axis_name="s"),
        compiler_params=pltpu.CompilerParams(needs_layout_passes=False),
        scratch_types=[
            pltpu.VMEM((CHUNK,), jnp.int32),
            pltpu.VMEM((CHUNK,), f32),
            pltpu.VMEM((CHUNK,), f32),
            pltpu.VMEM((CHUNK,), f32),
            pltpu.SemaphoreType.DMA,
        ],
    )
    y, exam, rel = call(x_flat, t_flat)
    return (
        jnp.reshape(y, (B, L)),
        jnp.reshape(exam, (B, L)),
        jnp.reshape(rel, (B, L)),
    )


# speedup vs baseline: 1.4328x; 1.3294x over previous
import jax
import jax.numpy as jnp
from jax import lax
from jax.experimental import pallas as pl
from jax.experimental.pallas import tpu as pltpu
from jax.experimental.pallas import tpu_sc as plsc

N_DOCS = 1000000
B = 16384
L = 50
NC = 2
NS = 16
NW = NC * NS
ROWS = B // NW
HALF = ROWS // 2
HN = HALF * L


def _body(x_hbm, table_hbm, y_hbm, exam_hbm, rel_hbm,
          idx_v, idxt_v, emb_v, y_v, exam_v, rel_v, sem):
    wid = lax.axis_index("s") * NC + lax.axis_index("c")
    iota = lax.iota(jnp.int32, 16)

    for h in range(2):
        base = wid * (2 * HN) + h * HN
        col0 = wid * ROWS + h * HALF

        pltpu.sync_copy(x_hbm.at[pl.ds(base, HN)], idx_v)

        def perm(l, carry):
            for b0 in range(0, HALF, 16):
                src = (b0 + iota) * L + l
                v = plsc.load_gather(idx_v, [src])
                idxt_v[pl.ds(l * HALF + b0, 16)] = v
            return carry

        lax.fori_loop(0, L, perm, 0)

        pltpu.async_copy(table_hbm.at[idxt_v], emb_v, sem).wait()

        def casc(c, carry):
            b0 = c * 16
            ex = jnp.full((16,), 1.0, dtype=jnp.float32)
            for l in range(L):
                e = emb_v[pl.ds(l * HALF + b0, 16)]
                r = 1.0 / (1.0 + jnp.exp(-e))
                rel_v[l, pl.ds(b0, 16)] = r
                exam_v[l, pl.ds(b0, 16)] = ex
                y_v[l, pl.ds(b0, 16)] = ex * r
                ex = ex * (1.0 - r)
            return carry

        lax.fori_loop(0, HALF // 16, casc, 0)

        pltpu.sync_copy(y_v, y_hbm.at[:, pl.ds(col0, HALF)])
        pltpu.sync_copy(exam_v, exam_hbm.at[:, pl.ds(col0, HALF)])
        pltpu.sync_copy(rel_v, rel_hbm.at[:, pl.ds(col0, HALF)])


def kernel(x, table):
    f32 = jnp.float32
    call = pl.kernel(
        _body,
        out_type=(
            jax.ShapeDtypeStruct((L, B), f32),
            jax.ShapeDtypeStruct((L, B), f32),
            jax.ShapeDtypeStruct((L, B), f32),
        ),
        mesh=plsc.VectorSubcoreMesh(core_axis_name="c", subcore_axis_name="s"),
        compiler_params=pltpu.CompilerParams(needs_layout_passes=False),
        scratch_types=[
            pltpu.VMEM((HN,), jnp.int32),
            pltpu.VMEM((HN,), jnp.int32),
            pltpu.VMEM((HN,), f32),
            pltpu.VMEM((L, HALF), f32),
            pltpu.VMEM((L, HALF), f32),
            pltpu.VMEM((L, HALF), f32),
            pltpu.SemaphoreType.DMA,
        ],
    )
    x_flat = jnp.reshape(x.astype(jnp.int32), (B * L,))
    t_flat = jnp.reshape(table, (N_DOCS,))
    y, exam, rel = call(x_flat, t_flat)
    return (
        jnp.swapaxes(y, 0, 1),
        jnp.swapaxes(exam, 0, 1),
        jnp.swapaxes(rel, 0, 1),
    )


# speedup vs baseline: 1.8180x; 1.2688x over previous
import jax
import jax.numpy as jnp
from jax import lax
from jax.experimental import pallas as pl
from jax.experimental.pallas import tpu as pltpu
from jax.experimental.pallas import tpu_sc as plsc

N_DOCS = 1000000
B = 16384
L = 50
NC = 2
NS = 16
NW = NC * NS
ROWS = B // NW
NQ = 4
QR = ROWS // NQ
QN = QR * L


def _body(x_hbm, table_hbm, y_hbm, exam_hbm, rel_hbm,
          idx_v, idxt0, idxt1, emb0, emb1,
          y0, ex0, rl0, y1, ex1, rl1, sg0, sg1, so0, so1):
    wid = lax.axis_index("s") * NC + lax.axis_index("c")
    iota = lax.iota(jnp.int32, 16)

    idxt = [idxt0, idxt1]
    emb = [emb0, emb1]
    outs = [(y0, ex0, rl0), (y1, ex1, rl1)]
    sg = [sg0, sg1]
    so = [so0, so1]

    def load_permute(q):
        tbuf = idxt[q % 2]
        base = wid * (NQ * QN) + q * QN
        pltpu.sync_copy(x_hbm.at[pl.ds(base, QN)], idx_v)

        def perm(l, carry):
            for b0 in range(0, QR, 16):
                src = (b0 + iota) * L + l
                tbuf[pl.ds(l * QR + b0, 16)] = plsc.load_gather(idx_v, [src])
            return carry

        lax.fori_loop(0, L, perm, 0)

    def start_gather(q):
        return pltpu.async_copy(table_hbm.at[idxt[q % 2]], emb[q % 2], sg[q % 2])

    gathers = [None] * NQ
    outcps = [None] * NQ

    load_permute(0)
    gathers[0] = start_gather(0)

    for q in range(NQ):
        nq = q + 1
        if nq < NQ:
            load_permute(nq)
            gathers[nq] = start_gather(nq)
        gathers[q].wait()
        if q >= 2:
            for cp in outcps[q - 2]:
                cp.wait()
        yb, eb, rb = outs[q % 2]
        ebuf = emb[q % 2]

        def casc(c, carry):
            b0 = c * 16
            ex = jnp.full((16,), 1.0, dtype=jnp.float32)
            for l in range(L):
                e = ebuf[pl.ds(l * QR + b0, 16)]
                r = 1.0 / (1.0 + jnp.exp(-e))
                rb[l, pl.ds(b0, 16)] = r
                eb[l, pl.ds(b0, 16)] = ex
                yb[l, pl.ds(b0, 16)] = ex * r
                ex = ex * (1.0 - r)
            return carry

        lax.fori_loop(0, QR // 16, casc, 0)

        col0 = wid * ROWS + q * QR
        sem = so[q % 2]
        outcps[q] = (
            pltpu.async_copy(yb, y_hbm.at[:, pl.ds(col0, QR)], sem),
            pltpu.async_copy(eb, exam_hbm.at[:, pl.ds(col0, QR)], sem),
            pltpu.async_copy(rb, rel_hbm.at[:, pl.ds(col0, QR)], sem),
        )

    for q in (NQ - 2, NQ - 1):
        for cp in outcps[q]:
            cp.wait()


def kernel(x, table):
    f32 = jnp.float32
    call = pl.kernel(
        _body,
        out_type=(
            jax.ShapeDtypeStruct((L, B), f32),
            jax.ShapeDtypeStruct((L, B), f32),
            jax.ShapeDtypeStruct((L, B), f32),
        ),
        mesh=plsc.VectorSubcoreMesh(core_axis_name="c", subcore_axis_name="s"),
        compiler_params=pltpu.CompilerParams(needs_layout_passes=False),
        scratch_types=[
            pltpu.VMEM((QN,), jnp.int32),
            pltpu.VMEM((QN,), jnp.int32),
            pltpu.VMEM((QN,), jnp.int32),
            pltpu.VMEM((QN,), f32),
            pltpu.VMEM((QN,), f32),
            pltpu.VMEM((L, QR), f32),
            pltpu.VMEM((L, QR), f32),
            pltpu.VMEM((L, QR), f32),
            pltpu.VMEM((L, QR), f32),
            pltpu.VMEM((L, QR), f32),
            pltpu.VMEM((L, QR), f32),
            pltpu.SemaphoreType.DMA,
            pltpu.SemaphoreType.DMA,
            pltpu.SemaphoreType.DMA,
            pltpu.SemaphoreType.DMA,
        ],
    )
    x_flat = jnp.reshape(x.astype(jnp.int32), (B * L,))
    t_flat = jnp.reshape(table, (N_DOCS,))
    y, exam, rel = call(x_flat, t_flat)
    return (
        jnp.swapaxes(y, 0, 1),
        jnp.swapaxes(exam, 0, 1),
        jnp.swapaxes(rel, 0, 1),
    )


# speedup vs baseline: 2.1537x; 1.1847x over previous
import jax
import jax.numpy as jnp
from jax import lax
from jax.experimental import pallas as pl
from jax.experimental.pallas import tpu as pltpu
from jax.experimental.pallas import tpu_sc as plsc

N_DOCS = 1000000
B = 16384
L = 50
NC = 2
NS = 16
NW = NC * NS
ROWS = B // NW
NQ = 4
QR = ROWS // NQ
QN = QR * L


def _body(x_hbm, table_hbm, y_hbm, exam_hbm, rel_hbm,
          idx_v, idxt0, idxt1, emb0, emb1,
          y0, ex0, rl0, y1, ex1, rl1, sg0, sg1, so0, so1):
    wid = lax.axis_index("s") * NC + lax.axis_index("c")
    iota = lax.iota(jnp.int32, 16)
    zeros = jnp.zeros((16,), jnp.int32)
    zeros = jnp.zeros((16,), jnp.int32)

    idxt = [idxt0, idxt1]
    emb = [emb0, emb1]
    outs = [(y0, ex0, rl0), (y1, ex1, rl1)]
    sg = [sg0, sg1]
    so = [so0, so1]

    def load_permute(q):
        tbuf = idxt[q % 2]
        col0 = wid * ROWS + q * QR
        pltpu.sync_copy(x_hbm.at[:, pl.ds(col0, QR)], idx_v)

        def perm(c, carry):
            b0 = c * 16
            for l in range(L):
                tbuf[pl.ds(l * QR + b0, 16)] = idx_v[l, pl.ds(b0, 16)]
            return carry

        lax.fori_loop(0, QR // 16, perm, 0)

    def start_gather(q):
        return pltpu.async_copy(table_hbm.at[idxt[q % 2]], emb[q % 2], sg[q % 2])

    gathers = [None] * NQ
    outcps = [None] * NQ

    load_permute(0)
    gathers[0] = start_gather(0)

    for q in range(NQ):
        nq = q + 1
        if nq < NQ:
            load_permute(nq)
            gathers[nq] = start_gather(nq)
        gathers[q].wait()
        if q >= 2:
            for cp in outcps[q - 2]:
                cp.wait()
        yb, eb, rb = outs[q % 2]
        ebuf = emb[q % 2]

        def casc(c, carry):
            b0 = c * 16
            ex = jnp.full((16,), 1.0, dtype=jnp.float32)
            for l in range(L):
                e = ebuf[pl.ds(l * QR + b0, 16)]
                r = 1.0 / (1.0 + jnp.exp(-e))
                rb[l, pl.ds(b0, 16)] = r
                eb[l, pl.ds(b0, 16)] = ex
                yb[l, pl.ds(b0, 16)] = ex * r
                ex = ex * (1.0 - r)
            return carry

        lax.fori_loop(0, QR // 16, casc, 0)

        col0 = wid * ROWS + q * QR
        sem = so[q % 2]
        outcps[q] = (
            pltpu.async_copy(yb, y_hbm.at[:, pl.ds(col0, QR)], sem),
            pltpu.async_copy(eb, exam_hbm.at[:, pl.ds(col0, QR)], sem),
            pltpu.async_copy(rb, rel_hbm.at[:, pl.ds(col0, QR)], sem),
        )

    for q in (NQ - 2, NQ - 1):
        for cp in outcps[q]:
            cp.wait()


def kernel(x, table):
    f32 = jnp.float32
    call = pl.kernel(
        _body,
        out_type=(
            jax.ShapeDtypeStruct((L, B), f32),
            jax.ShapeDtypeStruct((L, B), f32),
            jax.ShapeDtypeStruct((L, B), f32),
        ),
        mesh=plsc.VectorSubcoreMesh(core_axis_name="c", subcore_axis_name="s"),
        compiler_params=pltpu.CompilerParams(needs_layout_passes=False),
        scratch_types=[
            pltpu.VMEM((L, QR), jnp.int32),
            pltpu.VMEM((QN,), jnp.int32),
            pltpu.VMEM((QN,), jnp.int32),
            pltpu.VMEM((QN,), f32),
            pltpu.VMEM((QN,), f32),
            pltpu.VMEM((L, QR), f32),
            pltpu.VMEM((L, QR), f32),
            pltpu.VMEM((L, QR), f32),
            pltpu.VMEM((L, QR), f32),
            pltpu.VMEM((L, QR), f32),
            pltpu.VMEM((L, QR), f32),
            pltpu.SemaphoreType.DMA,
            pltpu.SemaphoreType.DMA,
            pltpu.SemaphoreType.DMA,
            pltpu.SemaphoreType.DMA,
        ],
    )
    t_flat = jnp.reshape(table, (N_DOCS,))
    xt = jnp.swapaxes(x.astype(jnp.int32), 0, 1)
    y, exam, rel = call(xt, t_flat)
    return (
        jnp.swapaxes(y, 0, 1),
        jnp.swapaxes(exam, 0, 1),
        jnp.swapaxes(rel, 0, 1),
    )
